# Initial kernel scaffold; baseline (speedup 1.0000x reference)
import jax
import jax.numpy as jnp
from jax.experimental import pallas as pl


def kernel(x, edge_index, edge_attr, W_conv, b_conv, W_ep, b_ep, W_gate, b_gate, ln_gamma, ln_beta):
    raise NotImplementedError("write your pallas kernel here")


# speedup vs baseline: 1.1502x; 1.1502x over previous
import functools

import jax
import jax.numpy as jnp
from jax import lax
from jax.experimental import pallas as pl
from jax.experimental.pallas import tpu as pltpu

N = 10000
E = 160000
D = 256
D_EDGE = 16


_BN = 1000


def _xlin_body(x_ref, w_ref, b_ref, out_ref):
    out_ref[...] = (
        jnp.dot(x_ref[...], w_ref[...], preferred_element_type=jnp.float32)
        + b_ref[...]
    )


def _xlin(x, W_conv, b_conv):
    return pl.pallas_call(
        _xlin_body,
        grid=(N // _BN,),
        in_specs=[
            pl.BlockSpec((_BN, D), lambda i: (i, 0)),
            pl.BlockSpec((D, D), lambda i: (0, 0)),
            pl.BlockSpec((1, D), lambda i: (0, 0)),
        ],
        out_specs=pl.BlockSpec((_BN, D), lambda i: (i, 0)),
        out_shape=jax.ShapeDtypeStruct((N, D), jnp.float32),
    )(x, W_conv, b_conv.reshape(1, D))


_BE = 2000


def _ef_body(ea_ref, w_ref, b_ref, ef_ref, ew_ref):
    ea = ea_ref[...]
    ef_ref[...] = jnp.maximum(
        jnp.dot(ea, w_ref[...], preferred_element_type=jnp.float32) + b_ref[...],
        0.0,
    )
    ew_ref[...] = jnp.sqrt(jnp.sum(ea * ea, axis=1, keepdims=True))


def _ef_and_weight(edge_attr, W_ep, b_ep):
    return pl.pallas_call(
        _ef_body,
        grid=(E // _BE,),
        in_specs=[
            pl.BlockSpec((_BE, D_EDGE), lambda i: (i, 0)),
            pl.BlockSpec((D_EDGE, D), lambda i: (0, 0)),
            pl.BlockSpec((1, D), lambda i: (0, 0)),
        ],
        out_specs=[
            pl.BlockSpec((_BE, D), lambda i: (i, 0)),
            pl.BlockSpec((_BE, 1), lambda i: (i, 0)),
        ],
        out_shape=[
            jax.ShapeDtypeStruct((E, D), jnp.float32),
            jax.ShapeDtypeStruct((E, 1), jnp.float32),
        ],
    )(edge_attr, W_ep, b_ep.reshape(1, D))


def _epilogue_body(xc_ref, ea_ref, xlin_ref, dinv_ref, wg1_ref, wg2_ref,
                   bg_ref, gamma_ref, beta_ref, out_ref):
    dinv = dinv_ref[...]
    x_conv = xc_ref[...] + (dinv * dinv) * xlin_ref[...]
    ef_agg = ea_ref[...]
    gate = jax.nn.sigmoid(
        jnp.dot(x_conv, wg1_ref[...], preferred_element_type=jnp.float32)
        + jnp.dot(ef_agg, wg2_ref[...], preferred_element_type=jnp.float32)
        + bg_ref[...]
    )
    out = gate * x_conv + (1.0 - gate) * ef_agg
    mu = jnp.mean(out, axis=-1, keepdims=True)
    var = jnp.mean((out - mu) ** 2, axis=-1, keepdims=True)
    out = (out - mu) * lax.rsqrt(var + 1e-5) * gamma_ref[...] + beta_ref[...]
    out_ref[...] = jnp.maximum(out, 0.0)


def _epilogue(x_conv, ef_agg, x_lin, dinv, W_gate, b_gate, ln_gamma, ln_beta):
    Wg1 = W_gate[:D]
    Wg2 = W_gate[D:]
    return pl.pallas_call(
        _epilogue_body,
        grid=(N // _BN,),
        in_specs=[
            pl.BlockSpec((_BN, D), lambda i: (i, 0)),
            pl.BlockSpec((_BN, D), lambda i: (i, 0)),
            pl.BlockSpec((_BN, D), lambda i: (i, 0)),
            pl.BlockSpec((_BN, 1), lambda i: (i, 0)),
            pl.BlockSpec((D, D), lambda i: (0, 0)),
            pl.BlockSpec((D, D), lambda i: (0, 0)),
            pl.BlockSpec((1, D), lambda i: (0, 0)),
            pl.BlockSpec((1, D), lambda i: (0, 0)),
            pl.BlockSpec((1, D), lambda i: (0, 0)),
        ],
        out_specs=pl.BlockSpec((_BN, D), lambda i: (i, 0)),
        out_shape=jax.ShapeDtypeStruct((N, D), jnp.float32),
    )(x_conv, ef_agg, x_lin, dinv.reshape(N, 1), Wg1, Wg2, b_gate.reshape(1, D),
      ln_gamma.reshape(1, D), ln_beta.reshape(1, D))


def kernel(x, edge_index, edge_attr, W_conv, b_conv, W_ep, b_ep, W_gate,
           b_gate, ln_gamma, ln_beta):
    src = edge_index[0]
    dst = edge_index[1]

    x_lin = _xlin(x, W_conv, b_conv)
    ef, edge_weight = _ef_and_weight(edge_attr, W_ep, b_ep)
    edge_weight = edge_weight.reshape(E)

    deg = jnp.ones((N,), jnp.float32).at[dst].add(edge_weight)
    dinv = lax.rsqrt(deg)

    norm = dinv[src] * edge_weight * dinv[dst]
    msg = norm[:, None] * x_lin[src]
    x_conv = jnp.zeros((N, D), jnp.float32).at[dst].add(msg)
    ef_agg = jnp.zeros((N, D), jnp.float32).at[src].add(ef)

    return _epilogue(x_conv, ef_agg, x_lin, dinv, W_gate, b_gate,
                     ln_gamma, ln_beta)
